# baseline (speedup 1.0000x reference)
import functools

import jax
import jax.numpy as jnp
from jax import lax
from jax.experimental import pallas as pl
from jax.experimental.pallas import tpu as pltpu
from jax.experimental.pallas import tpu_sc as plsc

_DIM = 64
_GAMMA = 12.0
_C = 128
_L = 16
_U = 8


@functools.lru_cache(maxsize=4)
def _build_sc_call(B, V_ent, V_rel):
    info = plsc.get_sparse_core_info()
    nw = info.num_cores * info.num_subcores
    rows_w = B // nw
    n_chunks = rows_w // _C
    assert rows_w % _C == 0 and B % nw == 0

    mesh = plsc.VectorSubcoreMesh(core_axis_name="c", subcore_axis_name="s")

    @functools.partial(
        pl.kernel,
        mesh=mesh,
        out_type=jax.ShapeDtypeStruct((B,), jnp.float32),
        compiler_params=pltpu.CompilerParams(
            needs_layout_passes=False, use_tc_tiling_on_sc=False),
        scratch_types=(
            [pltpu.VMEM((rows_w, 3), jnp.int32)]
            + [pltpu.VMEM((n_chunks, _C), jnp.int32) for _ in range(3)]
            + [pltpu.VMEM((2, _C, _DIM), jnp.float32) for _ in range(5)]
            + [pltpu.VMEM((_C,), jnp.float32),
               pltpu.SemaphoreType.DMA,
               pltpu.SemaphoreType.DMA]
        ),
    )
    def sc_call(ps_hbm, ent_hbm, rel_hbm, entp_hbm, out_hbm,
                ps_v, ih_v, ir_v, it_v, h_v, r_v, t_v, hp_v, tp_v, out_v,
                sem0, sem1):
        wid = lax.axis_index("s") * info.num_cores + lax.axis_index("c")
        base_w = wid * rows_w
        sems = (sem0, sem1)

        pltpu.sync_copy(ps_hbm.at[pl.ds(base_w, rows_w)], ps_v)
        iota = lax.iota(jnp.int32, _L)
        zi16 = jnp.zeros((_L,), jnp.int32)
        for g in range(rows_w // _L):
            rows = g * _L + iota
            k, off = g // (_C // _L), (g % (_C // _L)) * _L
            for c, buf in ((0, ih_v), (1, ir_v), (2, it_v)):
                v = plsc.load_gather(ps_v, [rows, zi16 + c])
                buf[k, pl.ds(off, _L)] = v

        def fire(k, slot):
            sem = sems[slot]
            return [
                pltpu.async_copy(ent_hbm.at[ih_v.at[k]], h_v.at[slot], sem),
                pltpu.async_copy(rel_hbm.at[ir_v.at[k]], r_v.at[slot], sem),
                pltpu.async_copy(ent_hbm.at[it_v.at[k]], t_v.at[slot], sem),
                pltpu.async_copy(entp_hbm.at[ih_v.at[k]], hp_v.at[slot], sem),
                pltpu.async_copy(entp_hbm.at[it_v.at[k]], tp_v.at[slot], sem),
            ]

        def compute(slot, k):
            hs, rs, ts = h_v.at[slot], r_v.at[slot], t_v.at[slot]
            hps, tps = hp_v.at[slot], tp_v.at[slot]

            def group(g, carry):
                rows = g * _L + lax.iota(jnp.int32, _L)
                zf = jnp.zeros((_L,), jnp.float32)
                zi = jnp.zeros((_L,), jnp.int32)

                def p1(j, c):
                    tph, tpt, rn, cols = c
                    for u in range(_U):
                        col = cols + u
                        tpv = plsc.load_gather(tps, [rows, col])
                        hv = plsc.load_gather(hs, [rows, col])
                        tv = plsc.load_gather(ts, [rows, col])
                        rv = plsc.load_gather(rs, [rows, col])
                        tph = tph + tpv * hv
                        tpt = tpt + tpv * tv
                        rn = rn + rv * rv
                    return tph, tpt, rn, cols + _U
                tph, tpt, rn, _ = lax.fori_loop(
                    0, _DIM // _U, p1, (zf, zf, zf, zi))

                dtp = tph - tpt
                x = jnp.maximum(rn, 1.0)
                yi = jnp.int32(0x5F3759DF) - (plsc.bitcast(x, jnp.int32) >> 1)
                y = plsc.bitcast(yi, jnp.float32)
                for _ in range(3):
                    y = y * (1.5 - 0.5 * x * y * y)
                nrm = x * y
                rscale = jnp.where(rn > 1.0, 1.0 / (nrm + 1e-7), 1.0)

                def p2(j, c):
                    acc, cols = c
                    for u in range(_U):
                        col = cols + u
                        hpv = plsc.load_gather(hps, [rows, col])
                        hv = plsc.load_gather(hs, [rows, col])
                        tv = plsc.load_gather(ts, [rows, col])
                        rv = plsc.load_gather(rs, [rows, col])
                        s = hpv * dtp + (hv - tv) + rv * rscale
                        acc = acc + jnp.abs(s)
                    return acc, cols + _U
                acc, _ = lax.fori_loop(0, _DIM // _U, p2, (zf, zi))

                out_v[pl.ds(g * _L, _L)] = acc - _GAMMA
                return carry
            lax.fori_loop(0, _C // _L, group, 0)
            pltpu.sync_copy(out_v, out_hbm.at[pl.ds(base_w + k * _C, _C)])

        pend = fire(0, 0)
        for k in range(n_chunks):
            nxt = fire(k + 1, (k + 1) % 2) if k + 1 < n_chunks else []
            for cp in pend:
                cp.wait()
            compute(k % 2, k)
            pend = nxt

    return sc_call


def kernel(pos_sample, ent_embd, rel_embd, ent_p):
    B = pos_sample.shape[0]
    sc_call = _build_sc_call(B, ent_embd.shape[0], rel_embd.shape[0])
    score = sc_call(pos_sample, ent_embd, rel_embd, ent_p)
    return score[:, None]


# speedup vs baseline: 1.1378x; 1.1378x over previous
import functools

import jax
import jax.numpy as jnp
from jax import lax
from jax.experimental import pallas as pl
from jax.experimental.pallas import tpu as pltpu
from jax.experimental.pallas import tpu_sc as plsc

_DIM = 64
_GAMMA = 12.0
_C = 128
_L = 16
_U = 8


@functools.lru_cache(maxsize=4)
def _build_sc_call(B):
    info = plsc.get_sparse_core_info()
    nw = info.num_cores * info.num_subcores
    rows_w = B // nw
    n_chunks = rows_w // _C
    assert rows_w % _C == 0 and B % nw == 0

    mesh = plsc.VectorSubcoreMesh(core_axis_name="c", subcore_axis_name="s")

    @functools.partial(
        pl.kernel,
        mesh=mesh,
        out_type=jax.ShapeDtypeStruct((B,), jnp.float32),
        compiler_params=pltpu.CompilerParams(
            needs_layout_passes=False, use_tc_tiling_on_sc=True),
        scratch_types=(
            [pltpu.VMEM((rows_w,), jnp.int32) for _ in range(3)]
            + [pltpu.VMEM((2, _C, 2 * _DIM), jnp.float32) for _ in range(3)]
            + [pltpu.VMEM((_C,), jnp.float32),
               pltpu.SemaphoreType.DMA,
               pltpu.SemaphoreType.DMA]
        ),
    )
    def sc_call(i0_hbm, i1_hbm, i2_hbm, cat_hbm, rel_hbm, out_hbm,
                i0_v, i1_v, i2_v, a_v, b_v, r_v, out_v, sem0, sem1):
        wid = lax.axis_index("s") * info.num_cores + lax.axis_index("c")
        base_w = wid * rows_w
        sems = (sem0, sem1)

        pltpu.sync_copy(i0_hbm.at[pl.ds(base_w, rows_w)], i0_v)
        pltpu.sync_copy(i1_hbm.at[pl.ds(base_w, rows_w)], i1_v)
        pltpu.sync_copy(i2_hbm.at[pl.ds(base_w, rows_w)], i2_v)

        def fire(k, slot):
            sem = sems[slot]
            sl = pl.ds(k * _C, _C)
            return [
                pltpu.async_copy(cat_hbm.at[i0_v.at[sl]], a_v.at[slot], sem),
                pltpu.async_copy(cat_hbm.at[i2_v.at[sl]], b_v.at[slot], sem),
                pltpu.async_copy(rel_hbm.at[i1_v.at[sl]], r_v.at[slot], sem),
            ]

        def compute(slot, k):
            a_s, b_s, r_s = a_v.at[slot], b_v.at[slot], r_v.at[slot]

            def group(g, carry):
                rows = g * _L + lax.iota(jnp.int32, _L)
                zf = jnp.zeros((_L,), jnp.float32)
                zi = jnp.zeros((_L,), jnp.int32)

                def p1(j, c):
                    tph, tpt, rn, cols = c
                    for u in range(_U):
                        col = cols + u
                        colp = col + _DIM
                        tpv = plsc.load_gather(b_s, [rows, colp])
                        hv = plsc.load_gather(a_s, [rows, col])
                        tv = plsc.load_gather(b_s, [rows, col])
                        rv = plsc.load_gather(r_s, [rows, col])
                        tph = tph + tpv * hv
                        tpt = tpt + tpv * tv
                        rn = rn + rv * rv
                    return tph, tpt, rn, cols + _U
                tph, tpt, rn, _ = lax.fori_loop(
                    0, _DIM // _U, p1, (zf, zf, zf, zi))

                dtp = tph - tpt
                x = jnp.maximum(rn, 1.0)
                yi = jnp.int32(0x5F3759DF) - (plsc.bitcast(x, jnp.int32) >> 1)
                y = plsc.bitcast(yi, jnp.float32)
                for _ in range(3):
                    y = y * (1.5 - 0.5 * x * y * y)
                nrm = x * y
                rscale = jnp.where(rn > 1.0, 1.0 / (nrm + 1e-7), 1.0)

                def p2(j, c):
                    acc, cols = c
                    for u in range(_U):
                        col = cols + u
                        colp = col + _DIM
                        hpv = plsc.load_gather(a_s, [rows, colp])
                        hv = plsc.load_gather(a_s, [rows, col])
                        tv = plsc.load_gather(b_s, [rows, col])
                        rv = plsc.load_gather(r_s, [rows, col])
                        s = hpv * dtp + (hv - tv) + rv * rscale
                        acc = acc + jnp.abs(s)
                    return acc, cols + _U
                acc, _ = lax.fori_loop(0, _DIM // _U, p2, (zf, zi))

                out_v[pl.ds(g * _L, _L)] = acc - _GAMMA
                return carry
            lax.fori_loop(0, _C // _L, group, 0)
            pltpu.sync_copy(out_v, out_hbm.at[pl.ds(base_w + k * _C, _C)])

        pend = fire(0, 0)
        for k in range(n_chunks):
            nxt = fire(k + 1, (k + 1) % 2) if k + 1 < n_chunks else []
            for cp in pend:
                cp.wait()
            compute(k % 2, k)
            pend = nxt

    return sc_call


def kernel(pos_sample, ent_embd, rel_embd, ent_p):
    B = pos_sample.shape[0]
    idx = pos_sample.astype(jnp.int32)
    cat = jnp.concatenate([ent_embd, ent_p], axis=1)
    rel128 = jnp.pad(rel_embd, ((0, 0), (0, _DIM)))
    sc_call = _build_sc_call(B)
    score = sc_call(idx[:, 0], idx[:, 1], idx[:, 2], cat, rel128)
    return score[:, None]


# speedup vs baseline: 1.2109x; 1.0642x over previous
import functools

import jax
import jax.numpy as jnp
from jax import lax
from jax.experimental import pallas as pl
from jax.experimental.pallas import tpu as pltpu
from jax.experimental.pallas import tpu_sc as plsc

_DIM = 64
_GAMMA = 12.0
_C = 128
_L = 16
_U = 8


@functools.lru_cache(maxsize=4)
def _build_sc_call(B):
    info = plsc.get_sparse_core_info()
    nw = info.num_cores * info.num_subcores
    rows_w = B // nw
    n_chunks = rows_w // _C
    assert rows_w % _C == 0 and B % nw == 0

    mesh = plsc.VectorSubcoreMesh(core_axis_name="c", subcore_axis_name="s")

    @functools.partial(
        pl.kernel,
        mesh=mesh,
        out_type=jax.ShapeDtypeStruct((B,), jnp.float32),
        compiler_params=pltpu.CompilerParams(
            needs_layout_passes=False, use_tc_tiling_on_sc=True),
        scratch_types=(
            [pltpu.VMEM((rows_w,), jnp.int32) for _ in range(4)]
            + [pltpu.VMEM((2, _C, 2 * _DIM), jnp.float32) for _ in range(3)]
            + [pltpu.VMEM((_C,), jnp.float32),
               pltpu.SemaphoreType.DMA,
               pltpu.SemaphoreType.DMA]
        ),
    )
    def sc_call(i0_hbm, i1_hbm, i2_hbm, cat_hbm, rel_hbm, out_hbm,
                i0_v, i1_v, i1h_v, i2_v, a_v, b_v, r_v, out_v, sem0, sem1):
        wid = lax.axis_index("s") * info.num_cores + lax.axis_index("c")
        base_w = wid * rows_w
        sems = (sem0, sem1)

        pltpu.sync_copy(i0_hbm.at[pl.ds(base_w, rows_w)], i0_v)
        pltpu.sync_copy(i1_hbm.at[pl.ds(base_w, rows_w)], i1_v)
        pltpu.sync_copy(i2_hbm.at[pl.ds(base_w, rows_w)], i2_v)

        def halve(j, c):
            v = i1_v[pl.ds(j * _L, _L)]
            i1h_v[pl.ds(j * _L, _L)] = v >> 1
            return c
        lax.fori_loop(0, rows_w // _L, halve, 0)

        def fire(k, slot):
            sem = sems[slot]
            sl = pl.ds(k * _C, _C)
            return [
                pltpu.async_copy(cat_hbm.at[i0_v.at[sl]], a_v.at[slot], sem),
                pltpu.async_copy(cat_hbm.at[i2_v.at[sl]], b_v.at[slot], sem),
                pltpu.async_copy(rel_hbm.at[i1h_v.at[sl]], r_v.at[slot], sem),
            ]

        def compute(slot, k):
            a_s, b_s, r_s = a_v.at[slot], b_v.at[slot], r_v.at[slot]

            def group(g, carry):
                rows = g * _L + lax.iota(jnp.int32, _L)
                zf = jnp.zeros((_L,), jnp.float32)
                zi = jnp.zeros((_L,), jnp.int32)
                i1g = i1_v[pl.ds(k * _C + g * _L, _L)]
                rbase = (i1g & 1) * _DIM

                def p1(j, c):
                    tph, tpt, rn, cols = c
                    for u in range(_U):
                        col = cols + u
                        colp = col + _DIM
                        tpv = plsc.load_gather(b_s, [rows, colp])
                        hv = plsc.load_gather(a_s, [rows, col])
                        tv = plsc.load_gather(b_s, [rows, col])
                        rv = plsc.load_gather(r_s, [rows, col + rbase])
                        tph = tph + tpv * hv
                        tpt = tpt + tpv * tv
                        rn = rn + rv * rv
                    return tph, tpt, rn, cols + _U
                tph, tpt, rn, _ = lax.fori_loop(
                    0, _DIM // _U, p1, (zf, zf, zf, zi))

                dtp = tph - tpt
                x = jnp.maximum(rn, 1.0)
                yi = jnp.int32(0x5F3759DF) - (plsc.bitcast(x, jnp.int32) >> 1)
                y = plsc.bitcast(yi, jnp.float32)
                for _ in range(3):
                    y = y * (1.5 - 0.5 * x * y * y)
                nrm = x * y
                rscale = jnp.where(rn > 1.0, 1.0 / (nrm + 1e-7), 1.0)

                def p2(j, c):
                    acc, cols = c
                    for u in range(_U):
                        col = cols + u
                        colp = col + _DIM
                        hpv = plsc.load_gather(a_s, [rows, colp])
                        hv = plsc.load_gather(a_s, [rows, col])
                        tv = plsc.load_gather(b_s, [rows, col])
                        rv = plsc.load_gather(r_s, [rows, col + rbase])
                        s = hpv * dtp + (hv - tv) + rv * rscale
                        acc = acc + jnp.abs(s)
                    return acc, cols + _U
                acc, _ = lax.fori_loop(0, _DIM // _U, p2, (zf, zi))

                out_v[pl.ds(g * _L, _L)] = acc - _GAMMA
                return carry
            lax.fori_loop(0, _C // _L, group, 0)
            pltpu.sync_copy(out_v, out_hbm.at[pl.ds(base_w + k * _C, _C)])

        pend = fire(0, 0)
        for k in range(n_chunks):
            nxt = fire(k + 1, (k + 1) % 2) if k + 1 < n_chunks else []
            for cp in pend:
                cp.wait()
            compute(k % 2, k)
            pend = nxt

    return sc_call


def kernel(pos_sample, ent_embd, rel_embd, ent_p):
    B = pos_sample.shape[0]
    idx = pos_sample.astype(jnp.int32)
    cat = jnp.concatenate([ent_embd, ent_p], axis=1)
    rel2 = rel_embd.reshape(rel_embd.shape[0] // 2, 2 * _DIM)
    sc_call = _build_sc_call(B)
    score = sc_call(idx[:, 0], idx[:, 1], idx[:, 2], cat, rel2)
    return score[:, None]
